# Initial kernel scaffold; baseline (speedup 1.0000x reference)
import jax
import jax.numpy as jnp
from jax.experimental import pallas as pl


def kernel(x, emb_weight):
    raise NotImplementedError("write your pallas kernel here")


# speedup vs baseline: 4.0799x; 4.0799x over previous
import functools

import jax
import jax.numpy as jnp
from jax import lax
from jax.experimental import pallas as pl
from jax.experimental.pallas import tpu as pltpu
from jax.experimental.pallas import tpu_sc as plsc

D = 64
NC, NS = 2, 16
NW = NC * NS
CH = 128


@functools.partial(jax.jit, static_argnums=(2,))
def _gather_rows(idx, table, b):
    nch = b // (NW * CH)
    mesh = plsc.VectorSubcoreMesh(core_axis_name="c", subcore_axis_name="s")

    @functools.partial(
        pl.kernel,
        out_type=jax.ShapeDtypeStruct((NW, nch, CH, D), jnp.float32),
        mesh=mesh,
        scratch_types=[
            pltpu.VMEM((nch, CH), jnp.int32),
            pltpu.VMEM((CH, D), jnp.float32),
            pltpu.SemaphoreType.DMA,
        ],
        compiler_params=pltpu.CompilerParams(use_tc_tiling_on_sc=False),
    )
    def k(idx_hbm, table_hbm, out_hbm, idx_v, buf, sem):
        wid = lax.axis_index("s") * NC + lax.axis_index("c")
        pltpu.sync_copy(idx_hbm.at[wid], idx_v)

        def chunk(c, carry):
            pltpu.async_copy(table_hbm.at[idx_v.at[c]], buf, sem).wait()
            pltpu.sync_copy(buf, out_hbm.at[wid, c])
            return carry

        lax.fori_loop(0, nch, chunk, 0, unroll=False)

    return k(idx, table)


def kernel(x, emb_weight):
    b = x.size
    idx = x.reshape(NW, b // (NW * CH), CH)
    out = _gather_rows(idx, emb_weight, b)
    return out.reshape(*x.shape, D)


# speedup vs baseline: 4.6503x; 1.1398x over previous
import functools

import jax
import jax.numpy as jnp
from jax import lax
from jax.experimental import pallas as pl
from jax.experimental.pallas import tpu as pltpu
from jax.experimental.pallas import tpu_sc as plsc

D = 64
NC, NS = 2, 16
NW = NC * NS
CH = 128


GB = 5


@functools.partial(jax.jit, static_argnums=(2,))
def _gather_rows(idx, table, b):
    nch = b // (NW * CH)
    ng = nch // GB
    mesh = plsc.VectorSubcoreMesh(core_axis_name="c", subcore_axis_name="s")

    @functools.partial(
        pl.kernel,
        out_type=jax.ShapeDtypeStruct((NW, nch, CH, D), jnp.float32),
        mesh=mesh,
        scratch_types=[
            pltpu.VMEM((nch, CH), jnp.int32),
            pltpu.VMEM((2, GB, CH, D), jnp.float32),
            pltpu.SemaphoreType.DMA,
            pltpu.SemaphoreType.DMA,
            pltpu.SemaphoreType.DMA,
            pltpu.SemaphoreType.DMA,
        ],
        compiler_params=pltpu.CompilerParams(use_tc_tiling_on_sc=False),
    )
    def k(idx_hbm, table_hbm, out_hbm, idx_v, buf, gs0, gs1, ws0, ws1):
        wid = lax.axis_index("s") * NC + lax.axis_index("c")
        pltpu.sync_copy(idx_hbm.at[wid], idx_v)
        gsems, wsems = (gs0, gs1), (ws0, ws1)

        def fire(g):
            p = g % 2
            return [
                pltpu.async_copy(
                    table_hbm.at[idx_v.at[g * GB + c]],
                    buf.at[p, c],
                    gsems[p],
                )
                for c in range(GB)
            ]

        gd = {0: fire(0)}
        wd = {}
        for g in range(ng):
            p = g % 2
            if g + 1 < ng:
                if g - 1 >= 0:
                    wd.pop(g - 1).wait()
                gd[g + 1] = fire(g + 1)
            for d in gd.pop(g):
                d.wait()
            wd[g] = pltpu.async_copy(
                buf.at[p],
                out_hbm.at[wid, pl.ds(g * GB, GB)],
                wsems[p],
            )
        for g in sorted(wd):
            wd.pop(g).wait()

    return k(idx, table)


def kernel(x, emb_weight):
    b = x.size
    idx = x.reshape(NW, b // (NW * CH), CH)
    out = _gather_rows(idx, emb_weight, b)
    return out.reshape(*x.shape, D)


# speedup vs baseline: 4.6548x; 1.0010x over previous
import functools

import jax
import jax.numpy as jnp
from jax import lax
from jax.experimental import pallas as pl
from jax.experimental.pallas import tpu as pltpu
from jax.experimental.pallas import tpu_sc as plsc

D = 64
NC, NS = 2, 16
NW = NC * NS
CH = 128


GB = 5
NBUF = 3


@functools.partial(jax.jit, static_argnums=(2,))
def _gather_rows(idx, table, b):
    nch = b // (NW * CH)
    ng = nch // GB
    mesh = plsc.VectorSubcoreMesh(core_axis_name="c", subcore_axis_name="s")

    @functools.partial(
        pl.kernel,
        out_type=jax.ShapeDtypeStruct((NW, nch, CH, D), jnp.float32),
        mesh=mesh,
        scratch_types=[
            pltpu.VMEM((nch, CH), jnp.int32),
            pltpu.VMEM((NBUF, GB, CH, D), jnp.float32),
        ] + [pltpu.SemaphoreType.DMA] * (2 * NBUF),
        compiler_params=pltpu.CompilerParams(use_tc_tiling_on_sc=False),
    )
    def k(idx_hbm, table_hbm, out_hbm, idx_v, buf, *sems):
        wid = lax.axis_index("s") * NC + lax.axis_index("c")
        pltpu.sync_copy(idx_hbm.at[wid], idx_v)
        gsems, wsems = sems[:NBUF], sems[NBUF:]

        def fire(g):
            p = g % NBUF
            return [
                pltpu.async_copy(
                    table_hbm.at[idx_v.at[g * GB + c]],
                    buf.at[p, c],
                    gsems[p],
                )
                for c in range(GB)
            ]

        gd = {g: fire(g) for g in range(min(NBUF - 1, ng))}
        wd = {}
        for g in range(ng):
            p = g % NBUF
            nxt = g + NBUF - 1
            if nxt < ng:
                prev = nxt - NBUF
                if prev >= 0:
                    wd.pop(prev).wait()
                gd[nxt] = fire(nxt)
            for d in gd.pop(g):
                d.wait()
            wd[g] = pltpu.async_copy(
                buf.at[p],
                out_hbm.at[wid, pl.ds(g * GB, GB)],
                wsems[p],
            )
        for g in sorted(wd):
            wd.pop(g).wait()

    return k(idx, table)


def kernel(x, emb_weight):
    b = x.size
    idx = x.reshape(NW, b // (NW * CH), CH)
    out = _gather_rows(idx, emb_weight, b)
    return out.reshape(*x.shape, D)
